# baseline (speedup 1.0000x reference)
import functools

import jax
import jax.numpy as jnp
from jax import lax
from jax.experimental import pallas as pl
from jax.experimental.pallas import tpu as pltpu
from jax.experimental.pallas import tpu_sc as plsc

F32 = jnp.float32
BF16 = jnp.bfloat16

_NC = 2
_NS = 16
_NW = _NC * _NS


def _pack_bf16_pair(a, v):
    a16 = jax.lax.bitcast_convert_type(a.astype(BF16), jnp.int16)
    v16 = jax.lax.bitcast_convert_type(v.astype(BF16), jnp.int16)
    return (a16.astype(jnp.int32) & 0xFFFF) | (v16.astype(jnp.int32) << 16)


def _unpack_bf16_pair(w):
    a = jax.lax.bitcast_convert_type(jax.lax.shift_left(w, 16), F32)
    v = jax.lax.bitcast_convert_type(w & jnp.int32(-65536), F32)
    return a, v


def _node_proj_body(x_ref, w_ref, b_ref, uh_ref, av_ref, bh_ref):
    r = jnp.dot(x_ref[...], w_ref[...], preferred_element_type=F32) + b_ref[...]
    uh_ref[...] = r[:, :128]
    av_ref[...] = _pack_bf16_pair(r[:, 128:256], r[:, 256:384])
    bh_ref[...] = r[:, 384:]


def _ln_block(x, g, b, eps=1e-5):
    m = jnp.mean(x, axis=-1, keepdims=True)
    v = jnp.mean(jnp.square(x - m), axis=-1, keepdims=True)
    return (x - m) / jnp.sqrt(v + eps) * g + b


def _edge_proj_body(e_ref, w_ref, b_ref, out_ref):
    out_ref[...] = (
        jnp.dot(e_ref[...], w_ref[...], preferred_element_type=F32) + b_ref[...]
    )


def _edge_elem_body(avd_ref, bhs_ref, ce_ref, e_ref, g_ref, b_ref,
                    msg_ref, eout_ref):
    a, v = _unpack_bf16_pair(avd_ref[...])
    en = a + bhs_ref[...] + ce_ref[...]
    gates = jax.nn.sigmoid(en)
    msg_ref[...] = gates * v
    e_norm = _ln_block(en, g_ref[...], b_ref[...])
    eout_ref[...] = e_ref[...] + jnp.maximum(e_norm, 0.0)


def _node_finish_body(h_ref, uh_ref, p0_ref, p1_ref, g_ref, b_ref, out_ref):
    s = uh_ref[...] + p0_ref[...] + p1_ref[...]
    h_new = _ln_block(s, g_ref[...], b_ref[...])
    out_ref[...] = h_ref[...] + jnp.maximum(h_new, 0.0)


def _sc_mesh():
    return plsc.VectorSubcoreMesh(
        core_axis_name="c", subcore_axis_name="s",
        num_cores=_NC, num_subcores=_NS)


def _make_gather(N, E, K):
    EW = E // _NW
    CH = EW // K
    assert CH % 2 == 1
    mesh = _sc_mesh()

    @functools.partial(
        pl.kernel,
        out_type=[
            jax.ShapeDtypeStruct((E, 128), jnp.int32),
            jax.ShapeDtypeStruct((E, 128), F32),
        ],
        mesh=mesh,
        scratch_types=[
            pltpu.VMEM((K,), jnp.int32),
            pltpu.VMEM((K,), jnp.int32),
            pltpu.VMEM((K,), jnp.int32),
            pltpu.VMEM((K,), jnp.int32),
            pltpu.VMEM((K, 128), jnp.int32),
            pltpu.VMEM((K, 128), jnp.int32),
            pltpu.VMEM((K, 128), F32),
            pltpu.VMEM((K, 128), F32),
            pltpu.SemaphoreType.DMA,
            pltpu.SemaphoreType.DMA,
            pltpu.SemaphoreType.DMA,
            pltpu.SemaphoreType.DMA,
            pltpu.SemaphoreType.DMA,
            pltpu.SemaphoreType.DMA,
            pltpu.SemaphoreType.DMA,
            pltpu.SemaphoreType.DMA,
            pltpu.SemaphoreType.DMA,
            pltpu.SemaphoreType.DMA,
            pltpu.SemaphoreType.DMA,
            pltpu.SemaphoreType.DMA,
        ],
        compiler_params=pltpu.CompilerParams(use_tc_tiling_on_sc=False),
    )
    def gather(av_hbm, bh_hbm, dst_hbm, src_hbm, avd_hbm, bhs_hbm,
               dstv0, dstv1, srcv0, srcv1, avrows0, avrows1, brows0, brows1,
               sd0, sd1, ss0, ss1, ga0, ga1, gb0, gb1, wa0, wa1, wb0, wb1):
        cid = lax.axis_index("c")
        sid = lax.axis_index("s")
        wid = sid * _NC + cid
        base = wid * EW
        dstvs = (dstv0, dstv1)
        srcvs = (srcv0, srcv1)
        avrowss = (avrows0, avrows1)
        browss = (brows0, brows1)
        sds = (sd0, sd1)
        sss = (ss0, ss1)
        gas = (ga0, ga1)
        gbs = (gb0, gb1)
        was = (wa0, wa1)
        wbs = (wb0, wb1)

        def idx_issue(c, b):
            off = base + c * K
            pltpu.async_copy(dst_hbm.at[pl.ds(off, K)], dstvs[b], sds[b])
            pltpu.async_copy(src_hbm.at[pl.ds(off, K)], srcvs[b], sss[b])

        def idx_wait(c, b):
            off = base + c * K
            pltpu.make_async_copy(
                dst_hbm.at[pl.ds(off, K)], dstvs[b], sds[b]).wait()
            pltpu.make_async_copy(
                src_hbm.at[pl.ds(off, K)], srcvs[b], sss[b]).wait()

        def gath_issue(b):
            pltpu.async_copy(av_hbm.at[dstvs[b]], avrowss[b], gas[b])
            pltpu.async_copy(bh_hbm.at[srcvs[b]], browss[b], gbs[b])

        def gath_wait(b):
            pltpu.make_async_copy(
                av_hbm.at[dstvs[b]], avrowss[b], gas[b]).wait()
            pltpu.make_async_copy(
                bh_hbm.at[srcvs[b]], browss[b], gbs[b]).wait()

        def wo_issue(c, b):
            off = base + c * K
            pltpu.async_copy(avrowss[b], avd_hbm.at[pl.ds(off, K)], was[b])
            pltpu.async_copy(browss[b], bhs_hbm.at[pl.ds(off, K)], wbs[b])

        def wo_wait(c, b):
            off = base + c * K
            pltpu.make_async_copy(
                avrowss[b], avd_hbm.at[pl.ds(off, K)], was[b]).wait()
            pltpu.make_async_copy(
                browss[b], bhs_hbm.at[pl.ds(off, K)], wbs[b]).wait()

        idx_issue(0, 0)
        idx_wait(0, 0)
        gath_issue(0)
        idx_issue(1, 1)

        def slot(c, b, first, last):
            nb = 1 - b
            gath_wait(b)
            wo_issue(c, b)
            if not last:
                idx_wait(c + 1, nb)
                if not first:
                    wo_wait(c - 1, nb)
                gath_issue(nb)

                @pl.when(c + 2 < CH)
                def _():
                    idx_issue(c + 2, b)

        slot(0, 0, first=True, last=False)

        def pair(cc, carry):
            for b in range(2):
                slot(cc * 2 + 1 + b, 1 - b, first=False, last=False)
            return carry

        lax.fori_loop(0, (CH - 3) // 2, pair, 0)
        slot(CH - 2, (CH - 2) % 2, first=False, last=False)
        slot(CH - 1, (CH - 1) % 2, first=False, last=True)
        wo_wait(CH - 2, (CH - 2) % 2)
        wo_wait(CH - 1, (CH - 1) % 2)

    return gather


def _make_scatter(N_pad, E, K):
    EW = E // _NW
    CH = EW // K
    assert CH % 2 == 1
    HALF = N_pad // 2
    NPT = HALF // _NS
    mesh = _sc_mesh()

    @functools.partial(
        pl.kernel,
        out_type=[
            jax.ShapeDtypeStruct((N_pad, 128), F32),
            jax.ShapeDtypeStruct((N_pad, 128), F32),
        ],
        mesh=mesh,
        scratch_types=[
            pltpu.VMEM_SHARED((HALF + 8, 128), F32),
            pltpu.VMEM((K,), jnp.int32),
            pltpu.VMEM((K,), jnp.int32),
            pltpu.VMEM((K,), jnp.int32),
            pltpu.VMEM((K,), jnp.int32),
            pltpu.VMEM((K, 128), F32),
            pltpu.VMEM((K, 128), F32),
            pltpu.VMEM((NPT, 128), F32),
            pltpu.SemaphoreType.DMA,
            pltpu.SemaphoreType.DMA,
            pltpu.SemaphoreType.DMA,
            pltpu.SemaphoreType.DMA,
        ],
        compiler_params=pltpu.CompilerParams(use_tc_tiling_on_sc=False),
    )
    def scatter(msg_hbm, src_hbm, zeros_hbm, p0_hbm, p1_hbm,
                agg_sh, srcv0, srcv1, idxv0, idxv1, mrows0, mrows1, obuf,
                si0, si1, sm0, sm1):
        cid = lax.axis_index("c")
        sid = lax.axis_index("s")
        wid = sid * _NC + cid
        base = wid * EW
        trash = jnp.full((16,), HALF, jnp.int32)
        srcvs = (srcv0, srcv1)
        idxvs = (idxv0, idxv1)
        mrowss = (mrows0, mrows1)
        sis = (si0, si1)
        sms = (sm0, sm1)

        def slot(c, b, lo, last):
            off = base + c * K
            pltpu.make_async_copy(
                src_hbm.at[pl.ds(off, K)], srcvs[b], sis[b]).wait()
            pltpu.make_async_copy(
                msg_hbm.at[pl.ds(off, K)], mrowss[b], sms[b]).wait()
            for j in range(K // 16):
                v = srcvs[b][pl.ds(j * 16, 16)] - lo
                ok = (v >= 0) & (v < HALF)
                idxvs[b][pl.ds(j * 16, 16)] = jnp.where(ok, v, trash)
            pltpu.sync_copy(mrowss[b], agg_sh.at[idxvs[b]], add=True)
            if not last:
                nxt = off + 2 * K

                @pl.when(c + 2 < CH)
                def _():
                    pltpu.async_copy(
                        src_hbm.at[pl.ds(nxt, K)], srcvs[b], sis[b])
                    pltpu.async_copy(
                        msg_hbm.at[pl.ds(nxt, K)], mrowss[b], sms[b])

        for p in range(2):
            lo = p * HALF

            @pl.when(sid == 0)
            def _():
                pltpu.sync_copy(zeros_hbm, agg_sh)

            plsc.subcore_barrier()

            for b in range(2):
                off = base + b * K
                pltpu.async_copy(src_hbm.at[pl.ds(off, K)], srcvs[b], sis[b])
                pltpu.async_copy(msg_hbm.at[pl.ds(off, K)], mrowss[b], sms[b])

            def pair(cc, carry):
                for b in range(2):
                    slot(cc * 2 + b, b, lo, last=False)
                return carry

            lax.fori_loop(0, CH // 2, pair, 0)
            slot(CH - 1, 0, lo, last=True)
            plsc.subcore_barrier()

            rows = sid * NPT
            pltpu.sync_copy(agg_sh.at[pl.ds(rows, NPT)], obuf)

            @pl.when(cid == 0)
            def _():
                pltpu.sync_copy(obuf, p0_hbm.at[pl.ds(lo + rows, NPT)])

            @pl.when(cid == 1)
            def _():
                pltpu.sync_copy(obuf, p1_hbm.at[pl.ds(lo + rows, NPT)])

            plsc.subcore_barrier()

    return scatter


def kernel(h, e, edge_index, U_w, U_b, V_w, V_b, A_w, A_b, B_w, B_b, C_w, C_b,
           ln_h_g, ln_h_b, ln_e_g, ln_e_b):
    N, H = h.shape
    E = e.shape[0]
    src = edge_index[0]
    dst = edge_index[1]

    BNP = 2000
    BN = 1000
    BE = 2000
    K = 80

    w_all = jnp.concatenate(
        [U_w.T, A_w.T, V_w.T, B_w.T], axis=1)
    b_all = jnp.concatenate([U_b, A_b, V_b, B_b]).reshape(1, 512)

    uh, av, bh = pl.pallas_call(
        _node_proj_body,
        grid=(N // BNP,),
        in_specs=[
            pl.BlockSpec((BNP, H), lambda i: (i, 0)),
            pl.BlockSpec((H, 4 * H), lambda i: (0, 0)),
            pl.BlockSpec((1, 4 * H), lambda i: (0, 0)),
        ],
        out_specs=[
            pl.BlockSpec((BNP, H), lambda i: (i, 0)),
            pl.BlockSpec((BNP, H), lambda i: (i, 0)),
            pl.BlockSpec((BNP, H), lambda i: (i, 0)),
        ],
        out_shape=[
            jax.ShapeDtypeStruct((N, H), F32),
            jax.ShapeDtypeStruct((N, H), jnp.int32),
            jax.ShapeDtypeStruct((N, H), F32),
        ],
    )(h, w_all, b_all)

    avd, bhs = _make_gather(N, E, K)(av, bh, dst, src)

    ce = pl.pallas_call(
        _edge_proj_body,
        grid=(E // BE,),
        in_specs=[
            pl.BlockSpec((BE, H), lambda i: (i, 0)),
            pl.BlockSpec((H, H), lambda i: (0, 0)),
            pl.BlockSpec((1, H), lambda i: (0, 0)),
        ],
        out_specs=pl.BlockSpec((BE, H), lambda i: (i, 0)),
        out_shape=jax.ShapeDtypeStruct((E, H), F32),
    )(e, C_w.T, C_b.reshape(1, H))

    msg, e_out = pl.pallas_call(
        _edge_elem_body,
        grid=(E // BE,),
        in_specs=[
            pl.BlockSpec((BE, H), lambda i: (i, 0)),
            pl.BlockSpec((BE, H), lambda i: (i, 0)),
            pl.BlockSpec((BE, H), lambda i: (i, 0)),
            pl.BlockSpec((BE, H), lambda i: (i, 0)),
            pl.BlockSpec((1, H), lambda i: (0, 0)),
            pl.BlockSpec((1, H), lambda i: (0, 0)),
        ],
        out_specs=[
            pl.BlockSpec((BE, H), lambda i: (i, 0)),
            pl.BlockSpec((BE, H), lambda i: (i, 0)),
        ],
        out_shape=[
            jax.ShapeDtypeStruct((E, H), F32),
            jax.ShapeDtypeStruct((E, H), F32),
        ],
    )(avd, bhs, ce, e, ln_e_g.reshape(1, H), ln_e_b.reshape(1, H))

    N_pad = 10240
    zeros = jnp.zeros((N_pad // 2 + 8, H), F32)
    p0, p1 = _make_scatter(N_pad, E, K)(msg, src, zeros)

    h_out = pl.pallas_call(
        _node_finish_body,
        grid=(N // BN,),
        in_specs=[
            pl.BlockSpec((BN, H), lambda i: (i, 0)),
            pl.BlockSpec((BN, H), lambda i: (i, 0)),
            pl.BlockSpec((BN, H), lambda i: (i, 0)),
            pl.BlockSpec((BN, H), lambda i: (i, 0)),
            pl.BlockSpec((1, H), lambda i: (0, 0)),
            pl.BlockSpec((1, H), lambda i: (0, 0)),
        ],
        out_specs=pl.BlockSpec((BN, H), lambda i: (i, 0)),
        out_shape=jax.ShapeDtypeStruct((N, H), F32),
    )(h, uh, p0, p1, ln_h_g.reshape(1, H), ln_h_b.reshape(1, H))

    return (h_out, e_out)


# speedup vs baseline: 1.1757x; 1.1757x over previous
import functools

import jax
import jax.numpy as jnp
from jax import lax
from jax.experimental import pallas as pl
from jax.experimental.pallas import tpu as pltpu
from jax.experimental.pallas import tpu_sc as plsc

F32 = jnp.float32
BF16 = jnp.bfloat16

_NC = 2
_NS = 16
_NW = _NC * _NS


def _pack_bf16_pair(a, v):
    a16 = jax.lax.bitcast_convert_type(a.astype(BF16), jnp.int16)
    v16 = jax.lax.bitcast_convert_type(v.astype(BF16), jnp.int16)
    return (a16.astype(jnp.int32) & 0xFFFF) | (v16.astype(jnp.int32) << 16)


def _unpack_bf16_pair(w):
    a = jax.lax.bitcast_convert_type(jax.lax.shift_left(w, 16), F32)
    v = jax.lax.bitcast_convert_type(w & jnp.int32(-65536), F32)
    return a, v


def _node_proj_body(x_ref, w_ref, b_ref, uh_ref, av_ref, bh_ref):
    r = jnp.dot(x_ref[...], w_ref[...], preferred_element_type=F32) + b_ref[...]
    uh_ref[...] = r[:, :128]
    av_ref[...] = _pack_bf16_pair(r[:, 128:256], r[:, 256:384])
    bh_ref[...] = r[:, 384:]


def _ln_block(x, g, b, eps=1e-5):
    m = jnp.mean(x, axis=-1, keepdims=True)
    v = jnp.mean(jnp.square(x - m), axis=-1, keepdims=True)
    return (x - m) / jnp.sqrt(v + eps) * g + b


def _edge_elem_body(avd_ref, bhs_ref, e_ref, cw_ref, cb_ref, g_ref, b_ref,
                    msg_ref, eout_ref):
    a, v = _unpack_bf16_pair(avd_ref[...])
    ce = jnp.dot(e_ref[...], cw_ref[...], preferred_element_type=F32) + cb_ref[...]
    en = a + bhs_ref[...] + ce
    gates = jax.nn.sigmoid(en)
    msg_ref[...] = gates * v
    e_norm = _ln_block(en, g_ref[...], b_ref[...])
    eout_ref[...] = e_ref[...] + jnp.maximum(e_norm, 0.0)


def _node_finish_body(h_ref, uh_ref, p0_ref, p1_ref, g_ref, b_ref, out_ref):
    s = uh_ref[...] + p0_ref[...] + p1_ref[...]
    h_new = _ln_block(s, g_ref[...], b_ref[...])
    out_ref[...] = h_ref[...] + jnp.maximum(h_new, 0.0)


def _sc_mesh():
    return plsc.VectorSubcoreMesh(
        core_axis_name="c", subcore_axis_name="s",
        num_cores=_NC, num_subcores=_NS)


def _make_gather(N, E, K):
    EW = E // _NW
    CH = EW // K
    assert CH % 2 == 1
    mesh = _sc_mesh()

    @functools.partial(
        pl.kernel,
        out_type=[
            jax.ShapeDtypeStruct((E, 128), jnp.int32),
            jax.ShapeDtypeStruct((E, 128), F32),
        ],
        mesh=mesh,
        scratch_types=[
            pltpu.VMEM((K,), jnp.int32),
            pltpu.VMEM((K,), jnp.int32),
            pltpu.VMEM((K,), jnp.int32),
            pltpu.VMEM((K,), jnp.int32),
            pltpu.VMEM((K, 128), jnp.int32),
            pltpu.VMEM((K, 128), jnp.int32),
            pltpu.VMEM((K, 128), F32),
            pltpu.VMEM((K, 128), F32),
            pltpu.SemaphoreType.DMA,
            pltpu.SemaphoreType.DMA,
            pltpu.SemaphoreType.DMA,
            pltpu.SemaphoreType.DMA,
            pltpu.SemaphoreType.DMA,
            pltpu.SemaphoreType.DMA,
            pltpu.SemaphoreType.DMA,
            pltpu.SemaphoreType.DMA,
            pltpu.SemaphoreType.DMA,
            pltpu.SemaphoreType.DMA,
            pltpu.SemaphoreType.DMA,
            pltpu.SemaphoreType.DMA,
        ],
        compiler_params=pltpu.CompilerParams(use_tc_tiling_on_sc=False),
    )
    def gather(av_hbm, bh_hbm, dst_hbm, src_hbm, avd_hbm, bhs_hbm,
               dstv0, dstv1, srcv0, srcv1, avrows0, avrows1, brows0, brows1,
               sd0, sd1, ss0, ss1, ga0, ga1, gb0, gb1, wa0, wa1, wb0, wb1):
        cid = lax.axis_index("c")
        sid = lax.axis_index("s")
        wid = sid * _NC + cid
        base = wid * EW
        dstvs = (dstv0, dstv1)
        srcvs = (srcv0, srcv1)
        avrowss = (avrows0, avrows1)
        browss = (brows0, brows1)
        sds = (sd0, sd1)
        sss = (ss0, ss1)
        gas = (ga0, ga1)
        gbs = (gb0, gb1)
        was = (wa0, wa1)
        wbs = (wb0, wb1)

        def idx_issue(c, b):
            off = base + c * K
            pltpu.async_copy(dst_hbm.at[pl.ds(off, K)], dstvs[b], sds[b])
            pltpu.async_copy(src_hbm.at[pl.ds(off, K)], srcvs[b], sss[b])

        def idx_wait(c, b):
            off = base + c * K
            pltpu.make_async_copy(
                dst_hbm.at[pl.ds(off, K)], dstvs[b], sds[b]).wait()
            pltpu.make_async_copy(
                src_hbm.at[pl.ds(off, K)], srcvs[b], sss[b]).wait()

        def gath_issue(b):
            pltpu.async_copy(av_hbm.at[dstvs[b]], avrowss[b], gas[b])
            pltpu.async_copy(bh_hbm.at[srcvs[b]], browss[b], gbs[b])

        def gath_wait(b):
            pltpu.make_async_copy(
                av_hbm.at[dstvs[b]], avrowss[b], gas[b]).wait()
            pltpu.make_async_copy(
                bh_hbm.at[srcvs[b]], browss[b], gbs[b]).wait()

        def wo_issue(c, b):
            off = base + c * K
            pltpu.async_copy(avrowss[b], avd_hbm.at[pl.ds(off, K)], was[b])
            pltpu.async_copy(browss[b], bhs_hbm.at[pl.ds(off, K)], wbs[b])

        def wo_wait(c, b):
            off = base + c * K
            pltpu.make_async_copy(
                avrowss[b], avd_hbm.at[pl.ds(off, K)], was[b]).wait()
            pltpu.make_async_copy(
                browss[b], bhs_hbm.at[pl.ds(off, K)], wbs[b]).wait()

        idx_issue(0, 0)
        idx_wait(0, 0)
        gath_issue(0)
        idx_issue(1, 1)

        def slot(c, b, first, last):
            nb = 1 - b
            gath_wait(b)
            wo_issue(c, b)
            if not last:
                idx_wait(c + 1, nb)
                if not first:
                    wo_wait(c - 1, nb)
                gath_issue(nb)

                @pl.when(c + 2 < CH)
                def _():
                    idx_issue(c + 2, b)

        slot(0, 0, first=True, last=False)

        def pair(cc, carry):
            for b in range(2):
                slot(cc * 2 + 1 + b, 1 - b, first=False, last=False)
            return carry

        lax.fori_loop(0, (CH - 3) // 2, pair, 0)
        slot(CH - 2, (CH - 2) % 2, first=False, last=False)
        slot(CH - 1, (CH - 1) % 2, first=False, last=True)
        wo_wait(CH - 2, (CH - 2) % 2)
        wo_wait(CH - 1, (CH - 1) % 2)

    return gather


def _make_scatter(N_pad, E, K):
    EW = E // _NW
    CH = EW // K
    assert CH % 2 == 1
    HALF = N_pad // 2
    NPT = HALF // _NS
    mesh = _sc_mesh()

    @functools.partial(
        pl.kernel,
        out_type=[
            jax.ShapeDtypeStruct((N_pad, 128), F32),
            jax.ShapeDtypeStruct((N_pad, 128), F32),
        ],
        mesh=mesh,
        scratch_types=[
            pltpu.VMEM_SHARED((HALF + 8, 128), F32),
            pltpu.VMEM((K,), jnp.int32),
            pltpu.VMEM((K,), jnp.int32),
            pltpu.VMEM((K,), jnp.int32),
            pltpu.VMEM((K,), jnp.int32),
            pltpu.VMEM((K, 128), F32),
            pltpu.VMEM((K, 128), F32),
            pltpu.VMEM((NPT, 128), F32),
            pltpu.SemaphoreType.DMA,
            pltpu.SemaphoreType.DMA,
            pltpu.SemaphoreType.DMA,
            pltpu.SemaphoreType.DMA,
        ],
        compiler_params=pltpu.CompilerParams(use_tc_tiling_on_sc=False),
    )
    def scatter(msg_hbm, src_hbm, zeros_hbm, p0_hbm, p1_hbm,
                agg_sh, srcv0, srcv1, idxv0, idxv1, mrows0, mrows1, obuf,
                si0, si1, sm0, sm1):
        cid = lax.axis_index("c")
        sid = lax.axis_index("s")
        wid = sid * _NC + cid
        base = wid * EW
        trash = jnp.full((16,), HALF, jnp.int32)
        srcvs = (srcv0, srcv1)
        idxvs = (idxv0, idxv1)
        mrowss = (mrows0, mrows1)
        sis = (si0, si1)
        sms = (sm0, sm1)

        def slot(c, b, lo, last):
            off = base + c * K
            pltpu.make_async_copy(
                src_hbm.at[pl.ds(off, K)], srcvs[b], sis[b]).wait()
            pltpu.make_async_copy(
                msg_hbm.at[pl.ds(off, K)], mrowss[b], sms[b]).wait()
            for j in range(K // 16):
                v = srcvs[b][pl.ds(j * 16, 16)] - lo
                ok = (v >= 0) & (v < HALF)
                idxvs[b][pl.ds(j * 16, 16)] = jnp.where(ok, v, trash)
            pltpu.sync_copy(mrowss[b], agg_sh.at[idxvs[b]], add=True)
            if not last:
                nxt = off + 2 * K

                @pl.when(c + 2 < CH)
                def _():
                    pltpu.async_copy(
                        src_hbm.at[pl.ds(nxt, K)], srcvs[b], sis[b])
                    pltpu.async_copy(
                        msg_hbm.at[pl.ds(nxt, K)], mrowss[b], sms[b])

        for p in range(2):
            lo = p * HALF

            @pl.when(sid == 0)
            def _():
                pltpu.sync_copy(zeros_hbm, agg_sh)

            plsc.subcore_barrier()

            for b in range(2):
                off = base + b * K
                pltpu.async_copy(src_hbm.at[pl.ds(off, K)], srcvs[b], sis[b])
                pltpu.async_copy(msg_hbm.at[pl.ds(off, K)], mrowss[b], sms[b])

            def pair(cc, carry):
                for b in range(2):
                    slot(cc * 2 + b, b, lo, last=False)
                return carry

            lax.fori_loop(0, CH // 2, pair, 0)
            slot(CH - 1, 0, lo, last=True)
            plsc.subcore_barrier()

            rows = sid * NPT
            pltpu.sync_copy(agg_sh.at[pl.ds(rows, NPT)], obuf)

            @pl.when(cid == 0)
            def _():
                pltpu.sync_copy(obuf, p0_hbm.at[pl.ds(lo + rows, NPT)])

            @pl.when(cid == 1)
            def _():
                pltpu.sync_copy(obuf, p1_hbm.at[pl.ds(lo + rows, NPT)])

            plsc.subcore_barrier()

    return scatter


def kernel(h, e, edge_index, U_w, U_b, V_w, V_b, A_w, A_b, B_w, B_b, C_w, C_b,
           ln_h_g, ln_h_b, ln_e_g, ln_e_b):
    N, H = h.shape
    E = e.shape[0]
    src = edge_index[0]
    dst = edge_index[1]

    BNP = 2000
    BN = 1000
    BE = 4000
    K = 80

    w_all = jnp.concatenate(
        [U_w.T, A_w.T, V_w.T, B_w.T], axis=1)
    b_all = jnp.concatenate([U_b, A_b, V_b, B_b]).reshape(1, 512)

    uh, av, bh = pl.pallas_call(
        _node_proj_body,
        grid=(N // BNP,),
        in_specs=[
            pl.BlockSpec((BNP, H), lambda i: (i, 0)),
            pl.BlockSpec((H, 4 * H), lambda i: (0, 0)),
            pl.BlockSpec((1, 4 * H), lambda i: (0, 0)),
        ],
        out_specs=[
            pl.BlockSpec((BNP, H), lambda i: (i, 0)),
            pl.BlockSpec((BNP, H), lambda i: (i, 0)),
            pl.BlockSpec((BNP, H), lambda i: (i, 0)),
        ],
        out_shape=[
            jax.ShapeDtypeStruct((N, H), F32),
            jax.ShapeDtypeStruct((N, H), jnp.int32),
            jax.ShapeDtypeStruct((N, H), F32),
        ],
    )(h, w_all, b_all)

    avd, bhs = _make_gather(N, E, K)(av, bh, dst, src)

    msg, e_out = pl.pallas_call(
        _edge_elem_body,
        grid=(E // BE,),
        in_specs=[
            pl.BlockSpec((BE, H), lambda i: (i, 0)),
            pl.BlockSpec((BE, H), lambda i: (i, 0)),
            pl.BlockSpec((BE, H), lambda i: (i, 0)),
            pl.BlockSpec((H, H), lambda i: (0, 0)),
            pl.BlockSpec((1, H), lambda i: (0, 0)),
            pl.BlockSpec((1, H), lambda i: (0, 0)),
            pl.BlockSpec((1, H), lambda i: (0, 0)),
        ],
        out_specs=[
            pl.BlockSpec((BE, H), lambda i: (i, 0)),
            pl.BlockSpec((BE, H), lambda i: (i, 0)),
        ],
        out_shape=[
            jax.ShapeDtypeStruct((E, H), F32),
            jax.ShapeDtypeStruct((E, H), F32),
        ],
    )(avd, bhs, e, C_w.T, C_b.reshape(1, H),
      ln_e_g.reshape(1, H), ln_e_b.reshape(1, H))

    N_pad = 10240
    zeros = jnp.zeros((N_pad // 2 + 8, H), F32)
    p0, p1 = _make_scatter(N_pad, E, K)(msg, src, zeros)

    h_out = pl.pallas_call(
        _node_finish_body,
        grid=(N // BN,),
        in_specs=[
            pl.BlockSpec((BN, H), lambda i: (i, 0)),
            pl.BlockSpec((BN, H), lambda i: (i, 0)),
            pl.BlockSpec((BN, H), lambda i: (i, 0)),
            pl.BlockSpec((BN, H), lambda i: (i, 0)),
            pl.BlockSpec((1, H), lambda i: (0, 0)),
            pl.BlockSpec((1, H), lambda i: (0, 0)),
        ],
        out_specs=pl.BlockSpec((BN, H), lambda i: (i, 0)),
        out_shape=jax.ShapeDtypeStruct((N, H), F32),
    )(h, uh, p0, p1, ln_h_g.reshape(1, H), ln_h_b.reshape(1, H))

    return (h_out, e_out)


# speedup vs baseline: 1.1945x; 1.0160x over previous
import functools

import jax
import jax.numpy as jnp
from jax import lax
from jax.experimental import pallas as pl
from jax.experimental.pallas import tpu as pltpu
from jax.experimental.pallas import tpu_sc as plsc

F32 = jnp.float32
BF16 = jnp.bfloat16

_NC = 2
_NS = 16
_NW = _NC * _NS


def _pack_bf16_pair(a, v):
    a16 = jax.lax.bitcast_convert_type(a.astype(BF16), jnp.int16)
    v16 = jax.lax.bitcast_convert_type(v.astype(BF16), jnp.int16)
    return (a16.astype(jnp.int32) & 0xFFFF) | (v16.astype(jnp.int32) << 16)


def _unpack_bf16_pair(w):
    a = jax.lax.bitcast_convert_type(jax.lax.shift_left(w, 16), F32)
    v = jax.lax.bitcast_convert_type(w & jnp.int32(-65536), F32)
    return a, v


def _node_proj_body(x_ref, w_ref, b_ref, uh_ref, av_ref, bh_ref):
    r = jnp.dot(x_ref[...], w_ref[...], preferred_element_type=F32) + b_ref[...]
    uh_ref[...] = r[:, :128]
    av_ref[...] = _pack_bf16_pair(r[:, 128:256], r[:, 256:384])
    bh_ref[...] = r[:, 384:]


def _ln_block(x, g, b, eps=1e-5):
    m = jnp.mean(x, axis=-1, keepdims=True)
    v = jnp.mean(jnp.square(x - m), axis=-1, keepdims=True)
    return (x - m) / jnp.sqrt(v + eps) * g + b


def _edge_elem_body(avd_ref, bhs_ref, e_ref, cw_ref, cb_ref, g_ref, b_ref,
                    msg_ref, eout_ref):
    a, v = _unpack_bf16_pair(avd_ref[...])
    ce = jnp.dot(e_ref[...], cw_ref[...], preferred_element_type=F32) + cb_ref[...]
    en = a + bhs_ref[...] + ce
    gates = jax.nn.sigmoid(en)
    msg_ref[...] = gates * v
    e_norm = _ln_block(en, g_ref[...], b_ref[...])
    eout_ref[...] = e_ref[...] + jnp.maximum(e_norm, 0.0)


def _node_finish_body(h_ref, uh_ref, p0_ref, p1_ref, g_ref, b_ref, out_ref):
    s = uh_ref[...] + p0_ref[...] + p1_ref[...]
    h_new = _ln_block(s, g_ref[...], b_ref[...])
    out_ref[...] = h_ref[...] + jnp.maximum(h_new, 0.0)


def _sc_mesh():
    return plsc.VectorSubcoreMesh(
        core_axis_name="c", subcore_axis_name="s",
        num_cores=_NC, num_subcores=_NS)


def _make_gather(N, E, K):
    EW = E // _NW
    CH = EW // K
    assert CH % 2 == 1
    mesh = _sc_mesh()

    @functools.partial(
        pl.kernel,
        out_type=[
            jax.ShapeDtypeStruct((E, 128), jnp.int32),
            jax.ShapeDtypeStruct((E, 128), F32),
        ],
        mesh=mesh,
        scratch_types=[
            pltpu.VMEM((K,), jnp.int32),
            pltpu.VMEM((K,), jnp.int32),
            pltpu.VMEM((K,), jnp.int32),
            pltpu.VMEM((K,), jnp.int32),
            pltpu.VMEM((K, 128), jnp.int32),
            pltpu.VMEM((K, 128), jnp.int32),
            pltpu.VMEM((K, 128), F32),
            pltpu.VMEM((K, 128), F32),
            pltpu.SemaphoreType.DMA,
            pltpu.SemaphoreType.DMA,
            pltpu.SemaphoreType.DMA,
            pltpu.SemaphoreType.DMA,
            pltpu.SemaphoreType.DMA,
            pltpu.SemaphoreType.DMA,
            pltpu.SemaphoreType.DMA,
            pltpu.SemaphoreType.DMA,
            pltpu.SemaphoreType.DMA,
            pltpu.SemaphoreType.DMA,
            pltpu.SemaphoreType.DMA,
            pltpu.SemaphoreType.DMA,
        ],
        compiler_params=pltpu.CompilerParams(use_tc_tiling_on_sc=False),
    )
    def gather(av_hbm, bh_hbm, dst_hbm, src_hbm, avd_hbm, bhs_hbm,
               dstv0, dstv1, srcv0, srcv1, avrows0, avrows1, brows0, brows1,
               sd0, sd1, ss0, ss1, ga0, ga1, gb0, gb1, wa0, wa1, wb0, wb1):
        cid = lax.axis_index("c")
        sid = lax.axis_index("s")
        wid = sid * _NC + cid
        base = wid * EW
        dstvs = (dstv0, dstv1)
        srcvs = (srcv0, srcv1)
        avrowss = (avrows0, avrows1)
        browss = (brows0, brows1)
        sds = (sd0, sd1)
        sss = (ss0, ss1)
        gas = (ga0, ga1)
        gbs = (gb0, gb1)
        was = (wa0, wa1)
        wbs = (wb0, wb1)

        def idx_issue(c, b):
            off = base + c * K
            pltpu.async_copy(dst_hbm.at[pl.ds(off, K)], dstvs[b], sds[b])
            pltpu.async_copy(src_hbm.at[pl.ds(off, K)], srcvs[b], sss[b])

        def idx_wait(c, b):
            off = base + c * K
            pltpu.make_async_copy(
                dst_hbm.at[pl.ds(off, K)], dstvs[b], sds[b]).wait()
            pltpu.make_async_copy(
                src_hbm.at[pl.ds(off, K)], srcvs[b], sss[b]).wait()

        def gath_issue(b):
            pltpu.async_copy(av_hbm.at[dstvs[b]], avrowss[b], gas[b])
            pltpu.async_copy(bh_hbm.at[srcvs[b]], browss[b], gbs[b])

        def gath_wait(b):
            pltpu.make_async_copy(
                av_hbm.at[dstvs[b]], avrowss[b], gas[b]).wait()
            pltpu.make_async_copy(
                bh_hbm.at[srcvs[b]], browss[b], gbs[b]).wait()

        def wo_issue(c, b):
            off = base + c * K
            pltpu.async_copy(avrowss[b], avd_hbm.at[pl.ds(off, K)], was[b])
            pltpu.async_copy(browss[b], bhs_hbm.at[pl.ds(off, K)], wbs[b])

        def wo_wait(c, b):
            off = base + c * K
            pltpu.make_async_copy(
                avrowss[b], avd_hbm.at[pl.ds(off, K)], was[b]).wait()
            pltpu.make_async_copy(
                browss[b], bhs_hbm.at[pl.ds(off, K)], wbs[b]).wait()

        idx_issue(0, 0)
        idx_wait(0, 0)
        gath_issue(0)
        idx_issue(1, 1)

        def slot(c, b, first, last):
            nb = 1 - b
            gath_wait(b)
            wo_issue(c, b)
            if not last:
                idx_wait(c + 1, nb)
                if not first:
                    wo_wait(c - 1, nb)
                gath_issue(nb)

                @pl.when(c + 2 < CH)
                def _():
                    idx_issue(c + 2, b)

        slot(0, 0, first=True, last=False)

        def pair(cc, carry):
            for b in range(2):
                slot(cc * 2 + 1 + b, 1 - b, first=False, last=False)
            return carry

        lax.fori_loop(0, (CH - 3) // 2, pair, 0)
        slot(CH - 2, (CH - 2) % 2, first=False, last=False)
        slot(CH - 1, (CH - 1) % 2, first=False, last=True)
        wo_wait(CH - 2, (CH - 2) % 2)
        wo_wait(CH - 1, (CH - 1) % 2)

    return gather


def _make_scatter(N_pad, E, K):
    EW = E // _NW
    CH = EW // K
    assert CH % 2 == 1
    HALF = N_pad // 2
    NPT = HALF // _NS
    mesh = _sc_mesh()

    @functools.partial(
        pl.kernel,
        out_type=[
            jax.ShapeDtypeStruct((N_pad, 128), F32),
            jax.ShapeDtypeStruct((N_pad, 128), F32),
        ],
        mesh=mesh,
        scratch_types=[
            pltpu.VMEM_SHARED((HALF + 8, 128), F32),
            pltpu.VMEM((K,), jnp.int32),
            pltpu.VMEM((K,), jnp.int32),
            pltpu.VMEM((K,), jnp.int32),
            pltpu.VMEM((K,), jnp.int32),
            pltpu.VMEM((K, 128), F32),
            pltpu.VMEM((K, 128), F32),
            pltpu.VMEM((NPT, 128), F32),
            pltpu.SemaphoreType.DMA,
            pltpu.SemaphoreType.DMA,
            pltpu.SemaphoreType.DMA,
            pltpu.SemaphoreType.DMA,
        ],
        compiler_params=pltpu.CompilerParams(use_tc_tiling_on_sc=False),
    )
    def scatter(msg_hbm, src_hbm, zeros_hbm, p0_hbm, p1_hbm,
                agg_sh, srcv0, srcv1, idxv0, idxv1, mrows0, mrows1, obuf,
                si0, si1, sm0, sm1):
        cid = lax.axis_index("c")
        sid = lax.axis_index("s")
        wid = sid * _NC + cid
        base = wid * EW
        trash = jnp.full((16,), HALF, jnp.int32)
        srcvs = (srcv0, srcv1)
        idxvs = (idxv0, idxv1)
        mrowss = (mrows0, mrows1)
        sis = (si0, si1)
        sms = (sm0, sm1)

        def slot(c, b, lo, last):
            off = base + c * K
            pltpu.make_async_copy(
                src_hbm.at[pl.ds(off, K)], srcvs[b], sis[b]).wait()
            pltpu.make_async_copy(
                msg_hbm.at[pl.ds(off, K)], mrowss[b], sms[b]).wait()
            for j in range(K // 16):
                v = srcvs[b][pl.ds(j * 16, 16)] - lo
                ok = (v >= 0) & (v < HALF)
                idxvs[b][pl.ds(j * 16, 16)] = jnp.where(ok, v, trash)
            pltpu.sync_copy(mrowss[b], agg_sh.at[idxvs[b]], add=True)
            if not last:
                nxt = off + 2 * K

                @pl.when(c + 2 < CH)
                def _():
                    pltpu.async_copy(
                        src_hbm.at[pl.ds(nxt, K)], srcvs[b], sis[b])
                    pltpu.async_copy(
                        msg_hbm.at[pl.ds(nxt, K)], mrowss[b], sms[b])

        for p in range(2):
            lo = p * HALF

            @pl.when(sid == 0)
            def _():
                pltpu.sync_copy(zeros_hbm, agg_sh)

            plsc.subcore_barrier()

            for b in range(2):
                off = base + b * K
                pltpu.async_copy(src_hbm.at[pl.ds(off, K)], srcvs[b], sis[b])
                pltpu.async_copy(msg_hbm.at[pl.ds(off, K)], mrowss[b], sms[b])

            def pair(cc, carry):
                for b in range(2):
                    slot(cc * 2 + b, b, lo, last=False)
                return carry

            lax.fori_loop(0, CH // 2, pair, 0)
            slot(CH - 1, 0, lo, last=True)
            plsc.subcore_barrier()

            rows = sid * NPT
            pltpu.sync_copy(agg_sh.at[pl.ds(rows, NPT)], obuf)

            @pl.when(cid == 0)
            def _():
                pltpu.sync_copy(obuf, p0_hbm.at[pl.ds(lo + rows, NPT)])

            @pl.when(cid == 1)
            def _():
                pltpu.sync_copy(obuf, p1_hbm.at[pl.ds(lo + rows, NPT)])

            plsc.subcore_barrier()

    return scatter


def kernel(h, e, edge_index, U_w, U_b, V_w, V_b, A_w, A_b, B_w, B_b, C_w, C_b,
           ln_h_g, ln_h_b, ln_e_g, ln_e_b):
    N, H = h.shape
    E = e.shape[0]
    src = edge_index[0]
    dst = edge_index[1]

    BNP = 5000
    BN = 2000
    BE = 8000
    K = 80

    w_all = jnp.concatenate(
        [U_w.T, A_w.T, V_w.T, B_w.T], axis=1)
    b_all = jnp.concatenate([U_b, A_b, V_b, B_b]).reshape(1, 512)

    uh, av, bh = pl.pallas_call(
        _node_proj_body,
        grid=(N // BNP,),
        in_specs=[
            pl.BlockSpec((BNP, H), lambda i: (i, 0)),
            pl.BlockSpec((H, 4 * H), lambda i: (0, 0)),
            pl.BlockSpec((1, 4 * H), lambda i: (0, 0)),
        ],
        out_specs=[
            pl.BlockSpec((BNP, H), lambda i: (i, 0)),
            pl.BlockSpec((BNP, H), lambda i: (i, 0)),
            pl.BlockSpec((BNP, H), lambda i: (i, 0)),
        ],
        out_shape=[
            jax.ShapeDtypeStruct((N, H), F32),
            jax.ShapeDtypeStruct((N, H), jnp.int32),
            jax.ShapeDtypeStruct((N, H), F32),
        ],
    )(h, w_all, b_all)

    avd, bhs = _make_gather(N, E, K)(av, bh, dst, src)

    msg, e_out = pl.pallas_call(
        _edge_elem_body,
        grid=(E // BE,),
        in_specs=[
            pl.BlockSpec((BE, H), lambda i: (i, 0)),
            pl.BlockSpec((BE, H), lambda i: (i, 0)),
            pl.BlockSpec((BE, H), lambda i: (i, 0)),
            pl.BlockSpec((H, H), lambda i: (0, 0)),
            pl.BlockSpec((1, H), lambda i: (0, 0)),
            pl.BlockSpec((1, H), lambda i: (0, 0)),
            pl.BlockSpec((1, H), lambda i: (0, 0)),
        ],
        out_specs=[
            pl.BlockSpec((BE, H), lambda i: (i, 0)),
            pl.BlockSpec((BE, H), lambda i: (i, 0)),
        ],
        out_shape=[
            jax.ShapeDtypeStruct((E, H), F32),
            jax.ShapeDtypeStruct((E, H), F32),
        ],
    )(avd, bhs, e, C_w.T, C_b.reshape(1, H),
      ln_e_g.reshape(1, H), ln_e_b.reshape(1, H))

    N_pad = 10240
    zeros = jnp.zeros((N_pad // 2 + 8, H), F32)
    p0, p1 = _make_scatter(N_pad, E, K)(msg, src, zeros)

    h_out = pl.pallas_call(
        _node_finish_body,
        grid=(N // BN,),
        in_specs=[
            pl.BlockSpec((BN, H), lambda i: (i, 0)),
            pl.BlockSpec((BN, H), lambda i: (i, 0)),
            pl.BlockSpec((BN, H), lambda i: (i, 0)),
            pl.BlockSpec((BN, H), lambda i: (i, 0)),
            pl.BlockSpec((1, H), lambda i: (0, 0)),
            pl.BlockSpec((1, H), lambda i: (0, 0)),
        ],
        out_specs=pl.BlockSpec((BN, H), lambda i: (i, 0)),
        out_shape=jax.ShapeDtypeStruct((N, H), F32),
    )(h, uh, p0, p1, ln_h_g.reshape(1, H), ln_h_b.reshape(1, H))

    return (h_out, e_out)
